# Initial kernel scaffold; baseline (speedup 1.0000x reference)
import jax
import jax.numpy as jnp
from jax.experimental import pallas as pl


def kernel(input_ids, table):
    raise NotImplementedError("write your pallas kernel here")


# speedup vs baseline: 1.7068x; 1.7068x over previous
import functools

import jax
import jax.numpy as jnp
from jax import lax
from jax.experimental import pallas as pl
from jax.experimental.pallas import tpu as pltpu
from jax.experimental.pallas import tpu_sc as plsc

VOCAB = 151936
HIDDEN = 1536

NC = 2
NS = 16
NW = NC * NS

B_TOTAL = 4 * 4096
B_PER_W = B_TOTAL // NW
CHUNK = 16
NBUF = 4
NCHUNK = B_PER_W // CHUNK


def _gather_body(table_hbm, idx_hbm, out_hbm, idx_v, rows_v, gsem, osem):
  wid = lax.axis_index("s") * NC + lax.axis_index("c")
  base = wid * B_PER_W
  pltpu.sync_copy(idx_hbm.at[pl.ds(base, B_PER_W)], idx_v)

  gathers = [None] * NCHUNK
  outs = [None] * NCHUNK
  for i in range(NCHUNK):
    b = i % NBUF
    if i >= NBUF:
      outs[i - NBUF].wait()
    gathers[i] = pltpu.async_copy(
        table_hbm.at[idx_v.at[pl.ds(i * CHUNK, CHUNK)]], rows_v.at[b], gsem)
    if i >= 1:
      gathers[i - 1].wait()
      outs[i - 1] = pltpu.async_copy(
          rows_v.at[(i - 1) % NBUF],
          out_hbm.at[pl.ds(base + (i - 1) * CHUNK, CHUNK)], osem)
  gathers[NCHUNK - 1].wait()
  outs[NCHUNK - 1] = pltpu.async_copy(
      rows_v.at[(NCHUNK - 1) % NBUF],
      out_hbm.at[pl.ds(base + (NCHUNK - 1) * CHUNK, CHUNK)], osem)
  for i in range(NBUF):
    outs[NCHUNK - NBUF + i].wait()


@jax.jit
def _gather(table, idx):
  mesh = plsc.VectorSubcoreMesh(core_axis_name="c", subcore_axis_name="s")
  f = pl.kernel(
      _gather_body,
      out_type=jax.ShapeDtypeStruct((B_TOTAL, HIDDEN), jnp.float32),
      mesh=mesh,
      scratch_types=[
          pltpu.VMEM((B_PER_W,), jnp.int32),
          pltpu.VMEM((NBUF, CHUNK, HIDDEN), jnp.float32),
          pltpu.SemaphoreType.DMA,
          pltpu.SemaphoreType.DMA,
      ],
  )
  return f(table, idx)


def kernel(input_ids, table):
  ids = input_ids.reshape(-1).astype(jnp.int32)
  out = _gather(table, ids)
  return out.reshape(input_ids.shape + (HIDDEN,))
